# baseline (speedup 1.0000x reference)
import functools

import jax
import jax.numpy as jnp
from jax import lax
from jax.experimental import pallas as pl
from jax.experimental.pallas import tpu as pltpu
from jax.experimental.pallas import tpu_sc as plsc

_NC = 2
_NS = 16
_NW = _NC * _NS
_L = 16

_BATCH = 16384
_FEAT = 64
_BPW = _BATCH // _NW
_CHUNK = 128
_NCHUNK = _BPW // _CHUNK


def _cl_body(feat_hbm, lab_hbm, cent_hbm, out_hbm,
             idx_v, feat_v, rows_v, acc_v, sem):
    cid = lax.axis_index("c")
    sid = lax.axis_index("s")
    wid = sid * _NC + cid
    base = wid * _BPW

    pltpu.sync_copy(lab_hbm.at[pl.ds(wid * _NCHUNK, _NCHUNK)], idx_v)

    copies = []
    for j in range(_NCHUNK):
        copies.append(pltpu.async_copy(
            cent_hbm.at[idx_v.at[j]],
            rows_v.at[pl.ds(j * _CHUNK, _CHUNK)],
            sem))
    pltpu.sync_copy(feat_hbm.at[pl.ds(base, _BPW)], feat_v)
    for c in copies:
        c.wait()

    nvec = _FEAT // _L

    def body(i, accs):
        out = []
        for j in range(nvec):
            f = feat_v[i, pl.ds(j * _L, _L)]
            c = rows_v[i, pl.ds(j * _L, _L)]
            d = f - c
            out.append(accs[j] + d * d)
        return tuple(out)

    zero = jnp.zeros((_L,), jnp.float32)
    accs = lax.fori_loop(0, _BPW, body, (zero,) * nvec)
    acc_v[...] = (accs[0] + accs[1]) + (accs[2] + accs[3])
    pltpu.sync_copy(acc_v, out_hbm.at[wid])


@functools.partial(jax.jit, static_argnums=())
def _partials(features, labels2d, centers):
    mesh = plsc.VectorSubcoreMesh(core_axis_name="c", subcore_axis_name="s")
    k = functools.partial(
        pl.kernel,
        out_type=jax.ShapeDtypeStruct((_NW, _L), jnp.float32),
        mesh=mesh,
        scratch_types=[
            pltpu.VMEM((_NCHUNK, _CHUNK), jnp.int32),
            pltpu.VMEM((_BPW, _FEAT), jnp.float32),
            pltpu.VMEM((_BPW, _FEAT), jnp.float32),
            pltpu.VMEM((_L,), jnp.float32),
            pltpu.SemaphoreType.DMA,
        ],
        compiler_params=pltpu.CompilerParams(use_tc_tiling_on_sc=False),
    )(_cl_body)
    return k(features, labels2d, centers)


def kernel(features, labels, centers):
    batch = features.shape[0]
    labels2d = labels.astype(jnp.int32).reshape(batch // _CHUNK, _CHUNK)
    part = _partials(features, labels2d, centers)
    return jnp.sum(part) * (0.5 / batch)


# speedup vs baseline: 1.3277x; 1.3277x over previous
import functools

import jax
import jax.numpy as jnp
from jax import lax
from jax.experimental import pallas as pl
from jax.experimental.pallas import tpu as pltpu
from jax.experimental.pallas import tpu_sc as plsc

_NC = 2
_NS = 16
_NW = _NC * _NS
_L = 16

_BATCH = 16384
_FEAT = 64
_BPW = _BATCH // _NW
_PASS = 256
_NPASS = _BPW // _PASS


def _cl_body(feat_hbm, lab_hbm, cent_hbm, out_hbm,
             lab_v, feat_v, rows_v, acc_v, sem):
    cid = lax.axis_index("c")
    sid = lax.axis_index("s")
    wid = sid * _NC + cid
    base = wid * _BPW

    pltpu.sync_copy(lab_hbm.at[pl.ds(base, _BPW)], lab_v)

    nvec = _FEAT // _L
    zero = jnp.zeros((_L,), jnp.float32)

    def do_pass(h, accs):
        row0 = h * _PASS

        def issue(g, carry):
            labs = lab_v[pl.ds(row0 + g * _L, _L)]
            for k in range(_L):
                pltpu.async_copy(cent_hbm.at[pl.ds(labs[k], 1)],
                                 rows_v.at[pl.ds(g * _L + k, 1)], sem)
            return carry

        lax.fori_loop(0, _PASS // _L, issue, 0)
        pltpu.sync_copy(feat_hbm.at[pl.ds(base + row0, _PASS)], feat_v)
        pltpu.make_async_copy(cent_hbm.at[pl.ds(0, _PASS)], rows_v,
                              sem).wait()

        def body(i, accs):
            out = []
            for j in range(nvec):
                f = feat_v[i, pl.ds(j * _L, _L)]
                c = rows_v[i, pl.ds(j * _L, _L)]
                d = f - c
                out.append(accs[j] + d * d)
            return tuple(out)

        return lax.fori_loop(0, _PASS, body, accs, unroll=2)

    accs = lax.fori_loop(0, _NPASS, do_pass, (zero,) * nvec)
    acc_v[...] = (accs[0] + accs[1]) + (accs[2] + accs[3])
    pltpu.sync_copy(acc_v, out_hbm.at[wid])


@jax.jit
def _partials(features, labels, centers):
    mesh = plsc.VectorSubcoreMesh(core_axis_name="c", subcore_axis_name="s")
    k = functools.partial(
        pl.kernel,
        out_type=jax.ShapeDtypeStruct((_NW, _L), jnp.float32),
        mesh=mesh,
        scratch_types=[
            pltpu.VMEM((_BPW,), jnp.int32),
            pltpu.VMEM((_PASS, _FEAT), jnp.float32),
            pltpu.VMEM((_PASS, _FEAT), jnp.float32),
            pltpu.VMEM((_L,), jnp.float32),
            pltpu.SemaphoreType.DMA,
        ],
    )(_cl_body)
    return k(features, labels, centers)


def kernel(features, labels, centers):
    batch = features.shape[0]
    part = _partials(features, labels.astype(jnp.int32), centers)
    return jnp.sum(part) * (0.5 / batch)


# speedup vs baseline: 2.1598x; 1.6267x over previous
import functools

import jax
import jax.numpy as jnp
from jax import lax
from jax.experimental import pallas as pl
from jax.experimental.pallas import tpu as pltpu
from jax.experimental.pallas import tpu_sc as plsc

_NC = 2
_NS = 16
_NW = _NC * _NS
_L = 16

_BATCH = 16384
_FEAT = 64
_CLASSES = 100000
_FPW = _FEAT // _NW
_HALF = _BATCH // 2


def _cl_body(featT_hbm, lab_hbm, centT_hbm, out_hbm,
             lab_v, feat_v, col_v, acc_v, sem):
    cid = lax.axis_index("c")
    sid = lax.axis_index("s")
    wid = sid * _NC + cid

    pltpu.sync_copy(lab_hbm, lab_v)

    zero = jnp.zeros((_L,), jnp.float32)
    accs = (zero,) * 4

    for jj in range(_FPW):
        j = wid * _FPW + jj
        pltpu.sync_copy(centT_hbm.at[j], col_v)
        for h in range(2):
            pltpu.sync_copy(featT_hbm.at[j, pl.ds(h * _HALF, _HALF)], feat_v)

            def body(g, accs, h=h):
                out = []
                for u in range(4):
                    off = (g * 4 + u) * _L
                    idx = lab_v[pl.ds(h * _HALF + off, _L)]
                    c = plsc.load_gather(col_v, [idx])
                    f = feat_v[pl.ds(off, _L)]
                    d = f - c
                    out.append(accs[u] + d * d)
                return tuple(out)

            accs = lax.fori_loop(0, _HALF // (4 * _L), body, accs)

    acc_v[...] = (accs[0] + accs[1]) + (accs[2] + accs[3])
    pltpu.sync_copy(acc_v, out_hbm.at[wid])


@jax.jit
def _partials(featT, labels, centT):
    mesh = plsc.VectorSubcoreMesh(core_axis_name="c", subcore_axis_name="s")
    k = functools.partial(
        pl.kernel,
        out_type=jax.ShapeDtypeStruct((_NW, _L), jnp.float32),
        mesh=mesh,
        scratch_types=[
            pltpu.VMEM((_BATCH,), jnp.int32),
            pltpu.VMEM((_HALF,), jnp.float32),
            pltpu.VMEM((_CLASSES,), jnp.float32),
            pltpu.VMEM((_L,), jnp.float32),
            pltpu.SemaphoreType.DMA,
        ],
        compiler_params=pltpu.CompilerParams(needs_layout_passes=False),
    )(_cl_body)
    return k(featT, labels, centT)


def kernel(features, labels, centers):
    batch = features.shape[0]
    part = _partials(features.T, labels.astype(jnp.int32), centers.T)
    return jnp.sum(part) * (0.5 / batch)
